# baseline (speedup 1.0000x reference)
import functools

import jax
import jax.numpy as jnp
from jax import lax
from jax.experimental import pallas as pl
from jax.experimental.pallas import tpu as pltpu
from jax.experimental.pallas import tpu_sc as plsc

B, S, H = 4, 4096, 768
N = B * S
HV = H // 16
EPS = 1e-6

NC, NS = 2, 16
NW = NC * NS
TPW = N // NW
C = 32
NCHUNK = TPW // C

_MESH = plsc.VectorSubcoreMesh(core_axis_name="c", subcore_axis_name="s")


@functools.partial(
    pl.kernel,
    out_type=jax.ShapeDtypeStruct((N, H), jnp.float32),
    mesh=_MESH,
    scratch_types=[
        pltpu.VMEM((NCHUNK, C), jnp.int32),
        pltpu.VMEM((NCHUNK, C), jnp.int32),
        pltpu.VMEM((NCHUNK, C), jnp.int32),
        pltpu.VMEM((C, H), jnp.float32),
        pltpu.VMEM((C, H), jnp.float32),
        pltpu.VMEM((C, H), jnp.float32),
        pltpu.VMEM((H,), jnp.float32),
        pltpu.VMEM((H,), jnp.float32),
        pltpu.SemaphoreType.DMA,
        pltpu.SemaphoreType.DMA,
        pltpu.SemaphoreType.DMA,
    ],
)
def _sc_embed(tok_ids, pos_ids, seg_ids, tok_tab, pos_tab, seg_tab, scale_h,
              bias_h, out_hbm, tokidx_v, posidx_v, segidx_v, tokbuf, posbuf,
              segbuf, scale_v, bias_v, sem_t, sem_p, sem_s):
    wid = lax.axis_index("s") * NC + lax.axis_index("c")

    pltpu.sync_copy(tok_ids.at[wid], tokidx_v)
    pltpu.sync_copy(pos_ids.at[wid], posidx_v)
    pltpu.sync_copy(seg_ids.at[wid], segidx_v)
    pltpu.sync_copy(scale_h, scale_v)
    pltpu.sync_copy(bias_h, bias_v)

    inv_h = jnp.float32(1.0 / H)
    lane = lax.iota(jnp.int32, 16)

    def lanesum(v):
        dnums = lax.GatherDimensionNumbers(
            offset_dims=(), collapsed_slice_dims=(0,), start_index_map=(0,))
        for sh in (8, 4, 2, 1):
            idx = jnp.bitwise_xor(lane, sh)
            v = v + lax.gather(v, idx[:, None], dnums, slice_sizes=(1,),
                               mode=lax.GatherScatterMode.PROMISE_IN_BOUNDS)
        return v

    def chunk_body(ci, carry):
        cp_t = pltpu.async_copy(tok_tab.at[tokidx_v.at[ci]], tokbuf, sem_t)
        cp_p = pltpu.async_copy(pos_tab.at[posidx_v.at[ci]], posbuf, sem_p)
        cp_s = pltpu.async_copy(seg_tab.at[segidx_v.at[ci]], segbuf, sem_s)
        cp_t.wait()
        cp_p.wait()
        cp_s.wait()

        def tok_body(t, tc):
            acc = jnp.zeros((16,), jnp.float32)
            acc2 = jnp.zeros((16,), jnp.float32)
            for j in range(HV):
                sl = pl.ds(j * 16, 16)
                v = tokbuf[t, sl] + posbuf[t, sl] + segbuf[t, sl]
                tokbuf[t, sl] = v
                acc = acc + v
                acc2 = acc2 + v * v
            s1 = lanesum(acc)
            s2 = lanesum(acc2)
            mean = s1 * inv_h
            var = s2 * inv_h - mean * mean
            x = var + EPS
            i = lax.bitcast_convert_type(x, jnp.int32)
            y = lax.bitcast_convert_type(
                jnp.int32(0x5F3759DF) - lax.shift_right_arithmetic(i, 1),
                jnp.float32)
            for _ in range(3):
                y = y * (1.5 - 0.5 * x * y * y)
            for j in range(HV):
                sl = pl.ds(j * 16, 16)
                tokbuf[t, sl] = ((tokbuf[t, sl] - mean) * y * scale_v[sl]
                                 + bias_v[sl])
            return tc

        lax.fori_loop(0, C, tok_body, 0)
        base = wid * TPW + ci * C
        pltpu.sync_copy(tokbuf, out_hbm.at[pl.ds(base, C)])
        return carry

    lax.fori_loop(0, NCHUNK, chunk_body, 0)


def kernel(input_ids, position_ids, segment_ids, token_table, pos_table,
           seg_table, ln_scale, ln_bias):
    tok = input_ids.reshape(NW, NCHUNK, C).astype(jnp.int32)
    pos = position_ids.reshape(NW, NCHUNK, C).astype(jnp.int32)
    seg = segment_ids.reshape(NW, NCHUNK, C).astype(jnp.int32)
    out = _sc_embed(tok, pos, seg, token_table, pos_table, seg_table,
                    ln_scale, ln_bias)
    return out.reshape(B, S, H)


# speedup vs baseline: 1.0650x; 1.0650x over previous
import functools

import jax
import jax.numpy as jnp
from jax import lax
from jax.experimental import pallas as pl
from jax.experimental.pallas import tpu as pltpu
from jax.experimental.pallas import tpu_sc as plsc

B, S, H = 4, 4096, 768
N = B * S
HV = H // 16
EPS = 1e-6

NC, NS = 2, 16
NW = NC * NS
TPW = N // NW
C = 16
NCHUNK = TPW // C
NBUF = 2
NROUND = NCHUNK // NBUF

_MESH = plsc.VectorSubcoreMesh(core_axis_name="c", subcore_axis_name="s")


@functools.partial(
    pl.kernel,
    out_type=jax.ShapeDtypeStruct((N, H), jnp.float32),
    mesh=_MESH,
    scratch_types=[
        pltpu.VMEM((NCHUNK, C), jnp.int32),
        pltpu.VMEM((NCHUNK, C), jnp.int32),
        pltpu.VMEM((NCHUNK, C), jnp.int32),
        pltpu.VMEM((H,), jnp.float32),
        pltpu.VMEM((H,), jnp.float32),
        pltpu.VMEM((C, H), jnp.float32),
        pltpu.VMEM((C, H), jnp.float32),
        pltpu.VMEM((C, H), jnp.float32),
        pltpu.VMEM((C, H), jnp.float32),
        pltpu.VMEM((C, H), jnp.float32),
        pltpu.VMEM((C, H), jnp.float32),
        pltpu.VMEM((C, H), jnp.float32),
        pltpu.VMEM((C, H), jnp.float32),
        pltpu.SemaphoreType.DMA,
        pltpu.SemaphoreType.DMA,
        pltpu.SemaphoreType.DMA,
        pltpu.SemaphoreType.DMA,
        pltpu.SemaphoreType.DMA,
        pltpu.SemaphoreType.DMA,
        pltpu.SemaphoreType.DMA,
        pltpu.SemaphoreType.DMA,
    ],
)
def _sc_embed(tok_ids, pos_ids, seg_ids, tok_tab, pos_tab, seg_tab, scale_h,
              bias_h, out_hbm, tokidx, posidx, segidx, scale_v, bias_v,
              tokb0, tokb1, posb0, posb1, segb0, segb1, outb0, outb1,
              st0, st1, sp0, sp1, ss0, ss1, so0, so1):
    wid = lax.axis_index("s") * NC + lax.axis_index("c")

    pltpu.sync_copy(tok_ids.at[wid], tokidx)
    pltpu.sync_copy(pos_ids.at[wid], posidx)
    pltpu.sync_copy(seg_ids.at[wid], segidx)
    pltpu.sync_copy(scale_h, scale_v)
    pltpu.sync_copy(bias_h, bias_v)

    tokbs = (tokb0, tokb1)
    posbs = (posb0, posb1)
    segbs = (segb0, segb1)
    outbs = (outb0, outb1)
    sems_t = (st0, st1)
    sems_p = (sp0, sp1)
    sems_s = (ss0, ss1)
    sems_o = (so0, so1)

    inv_h = jnp.float32(1.0 / H)
    lane = lax.iota(jnp.int32, 16)
    dnums = lax.GatherDimensionNumbers(
        offset_dims=(), collapsed_slice_dims=(0,), start_index_map=(0,))

    def lanesum(v):
        for sh in (8, 4, 2, 1):
            idx = jnp.bitwise_xor(lane, sh)
            v = v + lax.gather(v, idx[:, None], dnums, slice_sizes=(1,),
                               mode=lax.GatherScatterMode.PROMISE_IN_BOUNDS)
        return v

    def compute(tokb, posb, segb, outb):
        def tok_body(t, tc):
            vs = []
            acc = jnp.zeros((16,), jnp.float32)
            acc2 = jnp.zeros((16,), jnp.float32)
            for j in range(HV):
                sl = pl.ds(j * 16, 16)
                v = tokb[t, sl] + posb[t, sl] + segb[t, sl]
                vs.append(v)
                acc = acc + v
                acc2 = acc2 + v * v
            mean = lanesum(acc) * inv_h
            var = lanesum(acc2) * inv_h - mean * mean
            x = var + EPS
            i = lax.bitcast_convert_type(x, jnp.int32)
            y = lax.bitcast_convert_type(
                jnp.int32(0x5F3759DF) - lax.shift_right_arithmetic(i, 1),
                jnp.float32)
            for _ in range(3):
                y = y * (1.5 - 0.5 * x * y * y)
            q = mean * y
            for j in range(HV):
                sl = pl.ds(j * 16, 16)
                outb[t, sl] = (vs[j] * y - q) * scale_v[sl] + bias_v[sl]
            return tc

        lax.fori_loop(0, C, tok_body, 0)

    def fire(b, ci):
        pltpu.async_copy(tok_tab.at[tokidx.at[ci]], tokbs[b], sems_t[b])
        pltpu.async_copy(pos_tab.at[posidx.at[ci]], posbs[b], sems_p[b])
        pltpu.async_copy(seg_tab.at[segidx.at[ci]], segbs[b], sems_s[b])

    for b in range(NBUF):
        fire(b, b)

    def round_body(r, rc):
        for b in range(NBUF):
            ci = r * NBUF + b
            pltpu.make_async_copy(
                tok_tab.at[tokidx.at[ci]], tokbs[b], sems_t[b]).wait()
            pltpu.make_async_copy(
                pos_tab.at[posidx.at[ci]], posbs[b], sems_p[b]).wait()
            pltpu.make_async_copy(
                seg_tab.at[segidx.at[ci]], segbs[b], sems_s[b]).wait()

            @pl.when(r > 0)
            def _drain():
                pltpu.make_async_copy(
                    outbs[b], out_hbm.at[pl.ds(0, C)], sems_o[b]).wait()

            compute(tokbs[b], posbs[b], segbs[b], outbs[b])
            base = wid * TPW + ci * C
            pltpu.async_copy(outbs[b], out_hbm.at[pl.ds(base, C)], sems_o[b])

            @pl.when(r < NROUND - 1)
            def _prefetch():
                fire(b, ci + NBUF)

        return rc

    lax.fori_loop(0, NROUND, round_body, 0)

    for b in range(NBUF):
        pltpu.make_async_copy(
            outbs[b], out_hbm.at[pl.ds(0, C)], sems_o[b]).wait()


def kernel(input_ids, position_ids, segment_ids, token_table, pos_table,
           seg_table, ln_scale, ln_bias):
    tok = input_ids.reshape(NW, NCHUNK, C).astype(jnp.int32)
    pos = position_ids.reshape(NW, NCHUNK, C).astype(jnp.int32)
    seg = segment_ids.reshape(NW, NCHUNK, C).astype(jnp.int32)
    out = _sc_embed(tok, pos, seg, token_table, pos_table, seg_table,
                    ln_scale, ln_bias)
    return out.reshape(B, S, H)
